# Initial kernel scaffold; baseline (speedup 1.0000x reference)
import jax
import jax.numpy as jnp
from jax.experimental import pallas as pl


def kernel(node_feats, edge_index, edge_weight, W_conv, alpha, gamma, beta, W_lin, W_cls):
    raise NotImplementedError("write your pallas kernel here")


# speedup vs baseline: 1.7188x; 1.7188x over previous
import functools

import jax
import jax.numpy as jnp
from jax import lax
from jax.experimental import pallas as pl
from jax.experimental.pallas import tpu as pltpu
from jax.experimental.pallas import tpu_sc as plsc

N = 10000
E = 320000
F = 128
NTILES = 32
EPT = E // NTILES
CH = 80
NCH = EPT // CH

_mesh = plsc.VectorSubcoreMesh(core_axis_name="c", subcore_axis_name="s")
_f32 = jnp.float32
_i32 = jnp.int32


@functools.partial(
    pl.kernel,
    out_type=jax.ShapeDtypeStruct((E, F), _f32),
    mesh=_mesh,
    scratch_types=[
        pltpu.VMEM((EPT,), _i32),
        pltpu.VMEM((EPT,), _f32),
        pltpu.VMEM((2, CH, F), _f32),
        pltpu.VMEM((CH,), _i32),
        pltpu.VMEM((CH,), _i32),
        pltpu.SemaphoreType.DMA,
        pltpu.SemaphoreType.DMA,
    ],
)
def _sc_gather_scale(h_hbm, src_hbm, w_hbm, msg_hbm,
                     srcb, wb, rows, si0, si1, sem0, sem1):
    c = lax.axis_index("c")
    s = lax.axis_index("s")
    wid = c * 16 + s
    base = wid * EPT
    sems = (sem0, sem1)
    sidx = (si0, si1)

    pltpu.sync_copy(src_hbm.at[pl.ds(base, EPT)], srcb)
    pltpu.sync_copy(w_hbm.at[pl.ds(base, EPT)], wb)

    def stage(j, b):
        for t in range(CH // 16):
            sidx[b][pl.ds(t * 16, 16)] = srcb[pl.ds(j * CH + t * 16, 16)]

    def start_gather(b):
        pltpu.async_copy(h_hbm.at[sidx[b]], rows.at[b], sems[b])

    def wait_gather(b):
        pltpu.make_async_copy(h_hbm.at[sidx[b]], rows.at[b], sems[b]).wait()

    def scale(j, b):
        def gbody(g, _):
            w16 = wb[pl.ds(j * CH + g * 16, 16)]
            for l in range(16):
                ws = jnp.take(w16, jnp.full((16,), l, _i32))
                r = g * 16 + l
                for t in range(F // 16):
                    sl = pl.ds(t * 16, 16)
                    rows[b, r, sl] = rows[b, r, sl] * ws
            return 0
        lax.fori_loop(0, CH // 16, gbody, 0)

    def flush(j, b):
        pltpu.sync_copy(rows.at[b], msg_hbm.at[pl.ds(base + j * CH, CH)])

    stage(0, 0)
    start_gather(0)

    def pair(i, _):
        a = 2 * i
        stage(a + 1, 1)
        wait_gather(0)
        start_gather(1)
        scale(a, 0)
        flush(a, 0)
        stage(a + 2, 0)
        start_gather(0)
        wait_gather(1)
        scale(a + 1, 1)
        flush(a + 1, 1)
        return 0
    lax.fori_loop(0, (NCH - 1) // 2, pair, 0)

    wait_gather(0)
    scale(NCH - 1, 0)
    flush(NCH - 1, 0)


def kernel(node_feats, edge_index, edge_weight, W_conv, alpha, gamma, beta,
           W_lin, W_cls):
    n = node_feats.shape[0]
    src = edge_index[0]
    dst = edge_index[1]
    out_deg = jnp.clip(jnp.bincount(src, length=n).astype(jnp.float32), 1.0, None)
    in_deg = jnp.clip(jnp.bincount(dst, length=n).astype(jnp.float32), 1.0, None)
    h = node_feats * (out_deg ** -0.5)[:, None]
    msg = _sc_gather_scale(h, src, edge_weight)
    agg = jax.ops.segment_sum(msg, dst, num_segments=n)
    h = agg * (in_deg ** -0.5)[:, None]
    h = h @ W_conv
    h = jax.nn.leaky_relu(h, 0.01)
    mean = jnp.mean(h, axis=0, keepdims=True)
    sub = h - alpha * mean
    var = jnp.mean(sub * sub, axis=0, keepdims=True)
    h = gamma * sub / jnp.sqrt(var + 1e-5) + beta
    g = jnp.mean(h, axis=0, keepdims=True)
    g = jax.nn.leaky_relu(g @ W_lin.T, 0.01)
    return g @ W_cls.T
